# baseline (speedup 1.0000x reference)
import functools

import jax
import jax.numpy as jnp
from jax import lax
from jax.experimental import pallas as pl
from jax.experimental.pallas import tpu as pltpu
from jax.experimental.pallas import tpu_sc as plsc

_NC, _NS = 2, 16
_NT = _NC * _NS
_K = 8
_C = 128
_BM = 256


def _mesh():
    return plsc.VectorSubcoreMesh(core_axis_name="c", subcore_axis_name="s")


def _sc_call(mode, base, hn, src3, dst3, npad, ch):
    half = npad // 2
    accn = half + 1024
    trash = half + 512
    zpt = accn // _NS
    opt = half // _NS

    @functools.partial(
        pl.kernel,
        out_type=jax.ShapeDtypeStruct((_NC, half, 128), jnp.float32),
        mesh=_mesh(),
        scratch_types=[
            pltpu.VMEM((ch, _C), jnp.int32),
            pltpu.VMEM((ch, _C), jnp.int32),
            pltpu.VMEM((_C, 128), jnp.float32),
            pltpu.VMEM((_C, 128), jnp.float32),
            pltpu.VMEM_SHARED((accn, 128), jnp.float32),
            pltpu.SemaphoreType.DMA,
            pltpu.SemaphoreType.DMA,
            pltpu.SemaphoreType.DMA,
            pltpu.SemaphoreType.DMA,
        ],
    )
    def k(hn_hbm, src_hbm, dst_hbm, out_hbm,
          src_v, dst_v, rows0, rows1, acc, g0, g1, s0, s1):
        co = lax.axis_index("c")
        s = lax.axis_index("s")
        blk = co * _NS + s
        pltpu.sync_copy(src_hbm.at[blk], src_v)
        pltpu.sync_copy(dst_hbm.at[blk], dst_v)

        @pl.loop(0, ch)
        def _x(i):
            @pl.loop(0, _C, step=16)
            def _x2(q):
                dl = dst_v[i, pl.ds(q, 16)] - base
                ok = (dl >= 0) & (dl < half)
                dst_v[i, pl.ds(q, 16)] = jnp.where(ok, dl, trash)

        @pl.loop(0, _C)
        def _z(i):
            @pl.loop(0, 128, step=16)
            def _z2(q):
                rows1[i, pl.ds(q, 16)] = jnp.zeros((16,), jnp.float32)

        @pl.loop(0, zpt, step=_C)
        def _zero(r):
            pltpu.sync_copy(rows1, acc.at[pl.ds(s * zpt + r, _C)])

        if mode == "deg":
            @pl.loop(0, _C)
            def _o(i):
                @pl.loop(0, 128, step=16)
                def _o2(q):
                    rows0[i, pl.ds(q, 16)] = jnp.ones((16,), jnp.float32)

            plsc.subcore_barrier()

            @pl.loop(0, ch)
            def _scat(j):
                pltpu.sync_copy(rows0, acc.at[dst_v.at[j]], add=True)
        else:
            pltpu.async_copy(hn_hbm.at[src_v.at[0]], rows0, g0)
            pltpu.async_copy(hn_hbm.at[src_v.at[1]], rows1, g1)
            plsc.subcore_barrier()

            @pl.loop(0, ch, step=2)
            def _main(j):
                pltpu.make_async_copy(hn_hbm.at[src_v.at[j]], rows0, g0).wait()
                pltpu.async_copy(rows0, acc.at[dst_v.at[j]], s0, add=True)

                pltpu.make_async_copy(hn_hbm.at[src_v.at[j + 1]], rows1,
                                      g1).wait()
                pltpu.async_copy(rows1, acc.at[dst_v.at[j + 1]], s1, add=True)

                pltpu.make_async_copy(rows0, acc.at[dst_v.at[j]], s0).wait()
                jn = lax.rem(j + 2, ch)
                pltpu.async_copy(hn_hbm.at[src_v.at[jn]], rows0, g0)

                pltpu.make_async_copy(rows1, acc.at[dst_v.at[j + 1]],
                                      s1).wait()
                jn1 = lax.rem(j + 3, ch)
                pltpu.async_copy(hn_hbm.at[src_v.at[jn1]], rows1, g1)

            pltpu.make_async_copy(hn_hbm.at[src_v.at[0]], rows0, g0).wait()
            pltpu.make_async_copy(hn_hbm.at[src_v.at[1]], rows1, g1).wait()

        plsc.subcore_barrier()
        pltpu.sync_copy(acc.at[pl.ds(s * opt, opt)],
                        out_hbm.at[co, pl.ds(s * opt, opt)])

    return k(hn, src3, dst3)


def _half_spec(npad):
    hb = npad // 2 // _BM
    return pl.BlockSpec((_NC, _BM, 128),
                        lambda i: (0, jnp.minimum(i, hb - 1), 0)), \
           pl.BlockSpec((_NC, _BM, 128),
                        lambda i: (0, jnp.maximum(i - hb, 0), 0)), hb


def _mm_body(hb, x_ref, w_ref, b_ref, dA_ref, dB_ref, h_ref, hn_ref, dv_ref):
    h = jnp.dot(x_ref[...], w_ref[...],
                preferred_element_type=jnp.float32) + b_ref[...]
    i = pl.program_id(0)
    degA = dA_ref[0, :, :1] + dA_ref[1, :, :1]
    degB = dB_ref[0, :, :1] + dB_ref[1, :, :1]
    deg = jnp.where(i < hb, degA, degB) + 1.0
    dv = jnp.broadcast_to(lax.rsqrt(deg), h.shape)
    h_ref[...] = h
    hn_ref[...] = h * dv
    dv_ref[...] = dv


def _mm_call(xp, w, b2, degA, degB, npad):
    f = jax.ShapeDtypeStruct((npad, 128), jnp.float32)
    sA, sB, hb = _half_spec(npad)
    return pl.pallas_call(
        functools.partial(_mm_body, hb),
        grid=(npad // _BM,),
        in_specs=[
            pl.BlockSpec((_BM, 128), lambda i: (i, 0)),
            pl.BlockSpec((128, 128), lambda i: (0, 0)),
            pl.BlockSpec((1, 128), lambda i: (0, 0)),
            sA, sB,
        ],
        out_specs=[pl.BlockSpec((_BM, 128), lambda i: (i, 0))] * 3,
        out_shape=[f, f, f],
    )(xp, w, b2, degA, degB)


def _comb_body(cl, dl, last, hb, pA_ref, pB_ref, hn_ref, h0_ref, dv_ref,
               h_ref, hno_ref):
    dv = dv_ref[...]
    i = pl.program_id(0)
    aggA = pA_ref[0] + pA_ref[1]
    aggB = pB_ref[0] + pB_ref[1]
    agg = jnp.where(i < hb, aggA, aggB) + hn_ref[...]
    h = cl * (agg * dv) + dl * h0_ref[...]
    h_ref[...] = h
    if last:
        hno_ref[...] = h
    else:
        hno_ref[...] = h * dv


def _comb_call(pA, pB, hn, h0, dvb, cl, dl, last, npad):
    f = jax.ShapeDtypeStruct((npad, 128), jnp.float32)
    sA, sB, hb = _half_spec(npad)
    return pl.pallas_call(
        functools.partial(_comb_body, cl, dl, last, hb),
        grid=(npad // _BM,),
        in_specs=[
            sA, sB,
            pl.BlockSpec((_BM, 128), lambda i: (i, 0)),
            pl.BlockSpec((_BM, 128), lambda i: (i, 0)),
            pl.BlockSpec((_BM, 128), lambda i: (i, 0)),
        ],
        out_specs=[pl.BlockSpec((_BM, 128), lambda i: (i, 0))] * 2,
        out_shape=[f, f],
    )(pA, pB, hn, h0, dvb)


def kernel(features, edge_index, W, b):
    n, d = features.shape
    e = edge_index.shape[1]
    npad = (n // 2560 + 1) * 2560
    half = npad // 2

    ept0 = -(-e // _NT)
    ch = -(-ept0 // _C)
    ch += ch % 2
    ept = ch * _C

    src = jnp.pad(edge_index[0], (0, _NT * ept - e)).reshape(_NT, ch, _C)
    dst = jnp.pad(edge_index[1], (0, _NT * ept - e),
                  constant_values=npad - 1).reshape(_NT, ch, _C)
    xp = jnp.pad(features, ((0, npad - n), (0, 0)))
    b2 = b.reshape(1, d)

    degA = _sc_call("deg", 0, xp, src, dst, npad, ch)
    degB = _sc_call("deg", half, xp, src, dst, npad, ch)
    h0, hn, dvb = _mm_call(xp, W, b2, degA, degB, npad)

    h = h0
    for l in range(1, _K + 1):
        pA = _sc_call("prop", 0, hn, src, dst, npad, ch)
        pB = _sc_call("prop", half, hn, src, dst, npad, ch)
        h, hn = _comb_call(pA, pB, hn, h0, dvb,
                           l / (l + 1.0), 1.0 / (l + 1.0), l == _K, npad)
    return h[:n]


# speedup vs baseline: 1.3742x; 1.3742x over previous
import functools

import jax
import jax.numpy as jnp
from jax import lax
from jax.experimental import pallas as pl
from jax.experimental.pallas import tpu as pltpu
from jax.experimental.pallas import tpu_sc as plsc

_NC, _NS = 2, 16
_NT = _NC * _NS
_K = 8
_C = 128
_BM = 256


def _mesh():
    return plsc.VectorSubcoreMesh(core_axis_name="c", subcore_axis_name="s")


def _part_call(srcF, dstF, npad, ch):
    half = npad // 2
    trash = half + 512
    ept = ch * _C
    nv = ept // 16

    @functools.partial(
        pl.kernel,
        out_type=[jax.ShapeDtypeStruct((2, _NT, ept), jnp.int32),
                  jax.ShapeDtypeStruct((2, _NT, ept), jnp.int32),
                  jax.ShapeDtypeStruct((2, _NT, 16), jnp.int32)],
        mesh=_mesh(),
        compiler_params=pltpu.CompilerParams(needs_layout_passes=False),
        scratch_types=[
            pltpu.VMEM((ept,), jnp.int32),
            pltpu.VMEM((ept,), jnp.int32),
            pltpu.VMEM((ept + 16,), jnp.int32),
            pltpu.VMEM((ept + 16,), jnp.int32),
            pltpu.VMEM((ept + 16,), jnp.int32),
            pltpu.VMEM((ept + 16,), jnp.int32),
            pltpu.VMEM((16,), jnp.int32),
        ],
    )
    def k(src_hbm, dst_hbm, srcL_hbm, dstL_hbm, cnt_hbm,
          siv, div, oAs, oAd, oBs, oBd, cv):
        co = lax.axis_index("c")
        s = lax.axis_index("s")
        blk = co * _NS + s
        pltpu.sync_copy(src_hbm.at[blk], siv)
        pltpu.sync_copy(dst_hbm.at[blk], div)

        zer = jnp.zeros((16,), jnp.int32)
        tra = jnp.full((16,), trash, jnp.int32)

        @pl.loop(0, ept + 16, step=16)
        def _pre(i):
            oAs[pl.ds(i, 16)] = zer
            oAd[pl.ds(i, 16)] = tra
            oBs[pl.ds(i, 16)] = zer
            oBd[pl.ds(i, 16)] = tra

        def body(j, carry):
            nA, nB = carry
            d = div[pl.ds(j * 16, 16)]
            sv = siv[pl.ds(j * 16, 16)]
            mA = d < half
            mB = jnp.logical_not(mA)
            plsc.store_compressed(oAs.at[pl.ds(nA, 16)], sv, mask=mA)
            plsc.store_compressed(oAd.at[pl.ds(nA, 16)], d, mask=mA)
            plsc.store_compressed(oBs.at[pl.ds(nB, 16)], sv, mask=mB)
            plsc.store_compressed(oBd.at[pl.ds(nB, 16)], d - half, mask=mB)
            nA = nA + jnp.max(plsc.all_reduce_population_count(mA))
            nB = nB + jnp.max(plsc.all_reduce_population_count(mB))
            return nA, nB

        nA, nB = lax.fori_loop(0, nv, body, (jnp.int32(0), jnp.int32(0)))

        i16 = lax.iota(jnp.int32, 16)
        cv[...] = jnp.where(i16 == 0, nA, 0)
        pltpu.sync_copy(cv, cnt_hbm.at[0, blk])
        cv[...] = jnp.where(i16 == 0, nB, 0)
        pltpu.sync_copy(cv, cnt_hbm.at[1, blk])
        pltpu.sync_copy(oAs.at[pl.ds(0, ept)], srcL_hbm.at[0, blk])
        pltpu.sync_copy(oAd.at[pl.ds(0, ept)], dstL_hbm.at[0, blk])
        pltpu.sync_copy(oBs.at[pl.ds(0, ept)], srcL_hbm.at[1, blk])
        pltpu.sync_copy(oBd.at[pl.ds(0, ept)], dstL_hbm.at[1, blk])

    return k(srcF, dstF)


def _sc_call(mode, hn, srcL, dstL, cnt, npad, ch):
    half = npad // 2
    accn = half + 1024
    zpt = accn // _NS
    opt = half // _NS

    @functools.partial(
        pl.kernel,
        out_type=jax.ShapeDtypeStruct((_NC, half, 128), jnp.float32),
        mesh=_mesh(),
        scratch_types=[
            pltpu.VMEM((ch, _C), jnp.int32),
            pltpu.VMEM((ch, _C), jnp.int32),
            pltpu.VMEM((_C, 128), jnp.float32),
            pltpu.VMEM((_C, 128), jnp.float32),
            pltpu.VMEM((16,), jnp.int32),
            pltpu.VMEM_SHARED((accn, 128), jnp.float32),
            pltpu.SemaphoreType.DMA,
            pltpu.SemaphoreType.DMA,
        ],
    )
    def k(hn_hbm, srcL_hbm, dstL_hbm, cnt_hbm, out_hbm,
          src_v, dst_v, rows0, rows1, cv, acc, g0, g1):
        co = lax.axis_index("c")
        s = lax.axis_index("s")

        @pl.loop(0, _C)
        def _z(i):
            @pl.loop(0, 128, step=16)
            def _z2(q):
                rows1[i, pl.ds(q, 16)] = jnp.zeros((16,), jnp.float32)

        @pl.loop(0, zpt, step=_C)
        def _zero(r):
            pltpu.sync_copy(rows1, acc.at[pl.ds(s * zpt + r, _C)])

        if mode == "deg":
            @pl.loop(0, _C)
            def _o(i):
                @pl.loop(0, 128, step=16)
                def _o2(q):
                    rows0[i, pl.ds(q, 16)] = jnp.ones((16,), jnp.float32)

        plsc.subcore_barrier()

        for slot in (0, 1):
            t = 2 * s + slot
            pltpu.sync_copy(dstL_hbm.at[co, t], dst_v)
            pltpu.sync_copy(cnt_hbm.at[co, t], cv)
            cnt = cv[...][0]
            nit = (cnt + 2 * _C - 1) // (2 * _C)

            if mode == "deg":
                def dbody(i, carry):
                    pltpu.sync_copy(rows0, acc.at[dst_v.at[2 * i]], add=True)
                    pltpu.sync_copy(rows0, acc.at[dst_v.at[2 * i + 1]],
                                    add=True)
                    return carry

                lax.fori_loop(0, nit, dbody, jnp.int32(0))
            else:
                pltpu.sync_copy(srcL_hbm.at[co, t], src_v)
                nch = 2 * nit
                pltpu.async_copy(hn_hbm.at[src_v.at[0]], rows0, g0)
                pltpu.async_copy(hn_hbm.at[src_v.at[1]], rows1, g1)

                def pbody(i, carry):
                    j = 2 * i
                    pltpu.make_async_copy(hn_hbm.at[src_v.at[j]], rows0,
                                          g0).wait()
                    pltpu.sync_copy(rows0, acc.at[dst_v.at[j]], add=True)
                    jn = lax.rem(j + 2, nch)
                    pltpu.async_copy(hn_hbm.at[src_v.at[jn]], rows0, g0)

                    pltpu.make_async_copy(hn_hbm.at[src_v.at[j + 1]], rows1,
                                          g1).wait()
                    pltpu.sync_copy(rows1, acc.at[dst_v.at[j + 1]], add=True)
                    jn1 = lax.rem(j + 3, nch)
                    pltpu.async_copy(hn_hbm.at[src_v.at[jn1]], rows1, g1)
                    return carry

                lax.fori_loop(0, nit, pbody, jnp.int32(0))

                pltpu.make_async_copy(hn_hbm.at[src_v.at[0]], rows0,
                                      g0).wait()
                pltpu.make_async_copy(hn_hbm.at[src_v.at[1]], rows1,
                                      g1).wait()

        plsc.subcore_barrier()
        pltpu.sync_copy(acc.at[pl.ds(s * opt, opt)],
                        out_hbm.at[co, pl.ds(s * opt, opt)])

    return k(hn, srcL, dstL, cnt)


def _half_specs(npad):
    hb = npad // 2 // _BM
    sA = pl.BlockSpec((1, _BM, 128), lambda i: (0, jnp.minimum(i, hb - 1), 0))
    sB = pl.BlockSpec((1, _BM, 128), lambda i: (1, jnp.maximum(i - hb, 0), 0))
    return sA, sB, hb


def _mm_body(hb, x_ref, w_ref, b_ref, dA_ref, dB_ref, h_ref, hn_ref, dv_ref):
    h = jnp.dot(x_ref[...], w_ref[...],
                preferred_element_type=jnp.float32) + b_ref[...]
    i = pl.program_id(0)
    deg = jnp.where(i < hb, dA_ref[0, :, :1], dB_ref[0, :, :1]) + 1.0
    dv = jnp.broadcast_to(lax.rsqrt(deg), h.shape)
    h_ref[...] = h
    hn_ref[...] = h * dv
    dv_ref[...] = dv


def _mm_call(xp, w, b2, degp, npad):
    f = jax.ShapeDtypeStruct((npad, 128), jnp.float32)
    sA, sB, hb = _half_specs(npad)
    return pl.pallas_call(
        functools.partial(_mm_body, hb),
        grid=(npad // _BM,),
        in_specs=[
            pl.BlockSpec((_BM, 128), lambda i: (i, 0)),
            pl.BlockSpec((128, 128), lambda i: (0, 0)),
            pl.BlockSpec((1, 128), lambda i: (0, 0)),
            sA, sB,
        ],
        out_specs=[pl.BlockSpec((_BM, 128), lambda i: (i, 0))] * 3,
        out_shape=[f, f, f],
    )(xp, w, b2, degp, degp)


def _comb_body(cl, dl, last, hb, pA_ref, pB_ref, hn_ref, h0_ref, dv_ref,
               h_ref, hno_ref):
    dv = dv_ref[...]
    i = pl.program_id(0)
    agg = jnp.where(i < hb, pA_ref[0], pB_ref[0]) + hn_ref[...]
    h = cl * (agg * dv) + dl * h0_ref[...]
    h_ref[...] = h
    if last:
        hno_ref[...] = h
    else:
        hno_ref[...] = h * dv


def _comb_call(p, hn, h0, dvb, cl, dl, last, npad):
    f = jax.ShapeDtypeStruct((npad, 128), jnp.float32)
    sA, sB, hb = _half_specs(npad)
    return pl.pallas_call(
        functools.partial(_comb_body, cl, dl, last, hb),
        grid=(npad // _BM,),
        in_specs=[
            sA, sB,
            pl.BlockSpec((_BM, 128), lambda i: (i, 0)),
            pl.BlockSpec((_BM, 128), lambda i: (i, 0)),
            pl.BlockSpec((_BM, 128), lambda i: (i, 0)),
        ],
        out_specs=[pl.BlockSpec((_BM, 128), lambda i: (i, 0))] * 2,
        out_shape=[f, f],
    )(p, p, hn, h0, dvb)


def kernel(features, edge_index, W, b):
    n, d = features.shape
    e = edge_index.shape[1]
    npad = (n // 2560 + 1) * 2560

    ept0 = -(-e // _NT)
    ch = -(-ept0 // _C)
    ch += ch % 2
    ept = ch * _C

    srcF = jnp.pad(edge_index[0], (0, _NT * ept - e)).reshape(_NT, ept)
    dstF = jnp.pad(edge_index[1], (0, _NT * ept - e),
                   constant_values=npad - 1).reshape(_NT, ept)
    xp = jnp.pad(features, ((0, npad - n), (0, 0)))
    b2 = b.reshape(1, d)

    srcL, dstL, cnt = _part_call(srcF, dstF, npad, ch)
    srcL = srcL.reshape(2, _NT, ch, _C)
    dstL = dstL.reshape(2, _NT, ch, _C)

    degp = _sc_call("deg", xp, srcL, dstL, cnt, npad, ch)
    h0, hn, dvb = _mm_call(xp, W, b2, degp, npad)

    h = h0
    for l in range(1, _K + 1):
        p = _sc_call("prop", hn, srcL, dstL, cnt, npad, ch)
        h, hn = _comb_call(p, hn, h0, dvb,
                           l / (l + 1.0), 1.0 / (l + 1.0), l == _K, npad)
    return h[:n]
